# baseline (speedup 1.0000x reference)
import functools
import math

import jax
import jax.numpy as jnp
import numpy as np
from jax import lax
from jax.experimental import pallas as pl
from jax.experimental.pallas import tpu as pltpu
from jax.experimental.pallas import tpu_sc as plsc

D = 64
SEQ = 50
L = 16
V = 1000000
VB = 999936
WV = 512
NWIN = 1954
NBP = 1968
FLAT = 204800
KPT = FLAT // 16
BKN = 221184
TRASH_B = 220000
OUT_ROWS = FLAT + 8
CHUNK = 128


def _pos_encoding(length: int, depth: int) -> np.ndarray:
    half = depth / 2
    positions = np.arange(length)[:, np.newaxis]
    depths = np.arange(half)[np.newaxis, :] / half
    angle_rates = 1 / 10000**depths
    angle_rads = positions * angle_rates
    return np.concatenate(
        [np.sin(angle_rads), np.cos(angle_rads)], axis=-1
    ).astype(np.float32)


_POS = _pos_encoding(SEQ, D)


@jax.jit
def _embed_sc(table_t, idx, pos1d, tail1d):
    mesh = plsc.VectorSubcoreMesh(core_axis_name="c", subcore_axis_name="s")

    @functools.partial(
        pl.kernel,
        out_type=jax.ShapeDtypeStruct((OUT_ROWS, 128), jnp.float32),
        mesh=mesh,
        scratch_types=[
            pltpu.VMEM((8, 4, 8, 128), jnp.float32),
            pltpu.VMEM((CHUNK, 128), jnp.float32),
            pltpu.VMEM((KPT,), jnp.int32),
            pltpu.VMEM((1600,), jnp.int32),
            pltpu.VMEM((1600,), jnp.int32),
            pltpu.VMEM((1600,), jnp.int32),
            pltpu.VMEM((NBP,), jnp.int32),
            pltpu.VMEM((NBP,), jnp.int32),
            pltpu.VMEM((NBP,), jnp.int32),
            pltpu.VMEM((NBP,), jnp.int32),
            pltpu.VMEM((NBP,), jnp.int32),
            pltpu.VMEM((NBP,), jnp.int32),
            pltpu.VMEM((SEQ * D,), jnp.float32),
            pltpu.VMEM(((V - VB) * D,), jnp.float32),
            pltpu.VMEM((CHUNK,), jnp.int32),
            pltpu.VMEM((CHUNK,), jnp.int32),
            pltpu.VMEM_SHARED((16, NBP), jnp.int32),
            pltpu.VMEM_SHARED((BKN,), jnp.int32),
            pltpu.VMEM_SHARED((BKN,), jnp.int32),
            pltpu.SemaphoreType.DMA,
        ],
        compiler_params=pltpu.CompilerParams(needs_layout_passes=False),
    )
    def body(tt_h, idx_h, pos_h, tail_h, out_h,
             slab, ostage, midx, sc_d, sc_i, sc_k,
             cnt, tot, ownp, starts, woff, rowb,
             posv, tailv, ci, ck, cnt_sh, bk_i, bk_k, sem):
        t = lax.axis_index("s")
        c = lax.axis_index("c")
        wid = t * 2 + c
        lane = lax.iota(jnp.int32, 16)

        pltpu.sync_copy(pos_h, posv)
        pltpu.sync_copy(tail_h, tailv)
        pltpu.sync_copy(idx_h.at[pl.ds(t * KPT, KPT)], midx)

        @pl.loop(0, NBP // L)
        def _(cb):
            z = jnp.zeros((L,), jnp.int32)
            cnt[pl.ds(cb * L, L)] = z
            tot[pl.ds(cb * L, L)] = z
            ownp[pl.ds(cb * L, L)] = z

        @pl.loop(0, KPT // L)
        def _(g):
            iv = midx[pl.ds(g * L, L)]
            wv = iv >> 9
            rank, last = plsc.scan_count(wv)
            plsc.addupdate_scatter(cnt, [wv], rank, mask=last)

        pltpu.sync_copy(cnt, cnt_sh.at[t])
        plsc.subcore_barrier()

        for tt in range(16):
            pltpu.sync_copy(cnt_sh.at[tt], rowb)

            @pl.loop(0, NBP // L)
            def _(cb):
                r = rowb[pl.ds(cb * L, L)]
                tot[pl.ds(cb * L, L)] += r
                ownp[pl.ds(cb * L, L)] += jnp.where(t > tt, r, 0)

        @pl.loop(0, NBP // L, init_carry=jnp.int32(0))
        def _(cb, carry):
            t8 = (tot[pl.ds(cb * L, L)] + 7) & ~7
            cum = plsc.cumsum(t8)
            starts[pl.ds(cb * L, L)] = cum - t8 + carry
            return carry + jnp.sum(t8)

        @pl.loop(0, NBP // L)
        def _(cb):
            woff[pl.ds(cb * L, L)] = starts[pl.ds(cb * L, L)] + ownp[pl.ds(cb * L, L)]

        for blk in range(8):
            @pl.loop(0, 100)
            def _(gg):
                g = blk * 100 + gg
                iv = midx[pl.ds(g * L, L)]
                wv = iv >> 9
                rank, last = plsc.scan_count(wv)
                dest = plsc.load_gather(woff, [wv]) + rank - 1
                y = g * L + lane
                s = y - 50 * ((y * 5243) >> 18)
                sc_d[pl.ds(gg * L, L)] = dest
                sc_i[pl.ds(gg * L, L)] = iv
                sc_k[pl.ds(gg * L, L)] = (t * KPT + y) | (s << 18)
                plsc.addupdate_scatter(woff, [wv], rank, mask=last)

            pltpu.sync_copy(sc_i, bk_i.at[sc_d])
            pltpu.sync_copy(sc_k, bk_k.at[sc_d])

        plsc.subcore_barrier()

        @pl.loop(0, 62)
        def _(jj):
            w = wid + 32 * jj

            @pl.when(w < NWIN)
            def _():
                ws = jnp.minimum(w, 1952)
                copies = []
                for o in range(8):
                    for j in range(4):
                        copies.append(pltpu.async_copy(
                            tt_h.at[pl.ds(o * 8, 8),
                                    pl.ds((ws * 4 + j) * 128, 128)],
                            slab.at[o, j], sem))
                for cp in copies:
                    cp.wait()

                wb = pl.multiple_of((w >> 4) << 4, L)
                sts = starts[pl.ds(wb, L)]
                tts = tot[pl.ds(wb, L)]
                sel = lane == (w - wb)
                st = pl.multiple_of(jnp.sum(jnp.where(sel, sts, 0)), 8)
                n = jnp.sum(jnp.where(sel, tts, 0))

                @pl.loop(0, (n + CHUNK - 1) >> 7)
                def _(ch):
                    pltpu.sync_copy(bk_i.at[pl.ds(st + ch * CHUNK, CHUNK)], ci)
                    pltpu.sync_copy(bk_k.at[pl.ds(st + ch * CHUNK, CHUNK)], ck)

                    @pl.loop(0, CHUNK // L)
                    def _(m):
                        iv = ci[pl.ds(m * L, L)]
                        kp = ck[pl.ds(m * L, L)]
                        kv = kp & 0x3FFFF
                        sv = jnp.minimum((kp >> 18) & 63, SEQ - 1)
                        valid = (m * L + lane) < (n - ch * CHUNK)
                        ivc = jnp.clip(iv, w * WV, w * WV + WV - 1)
                        ck[pl.ds(m * L, L)] = jnp.where(valid, kv, FLAT)
                        jv = (ivc - w * WV) >> 7
                        cv = ivc & 127
                        tl = jnp.clip(ivc - VB, 0, V - VB - 1)
                        rowv = m * L + lane
                        for o in range(8):
                            ov = jnp.full((L,), o, jnp.int32)
                            for r in range(8):
                                dd = o * 8 + r
                                pv = plsc.load_gather(posv, [sv * D + dd])

                                @pl.when(w < 1953)
                                def _():
                                    rv = jnp.full((L,), r, jnp.int32)
                                    tv = plsc.load_gather(slab, [ov, jv, rv, cv])
                                    plsc.store_scatter(
                                        ostage, [rowv, jnp.full((L,), dd, jnp.int32)],
                                        tv * 8.0 + pv)

                                @pl.when(w == 1953)
                                def _():
                                    tv = plsc.load_gather(tailv, [dd * (V - VB) + tl])
                                    plsc.store_scatter(
                                        ostage, [rowv, jnp.full((L,), dd, jnp.int32)],
                                        tv * 8.0 + pv)

                    pltpu.sync_copy(ostage, out_h.at[ck])

    return body(table_t, idx, pos1d, tail1d)


def kernel(x, table):
    batch, seq = x.shape
    assert seq == SEQ and table.shape == (V, D)
    idx = x.reshape(-1).astype(jnp.int32)
    table_t = table.T
    tail1d = table[VB:, :].T.reshape(-1)
    pos1d = jnp.asarray(_POS).reshape(-1)
    out128 = _embed_sc(table_t, idx, pos1d, tail1d)
    return out128[:FLAT, :D].reshape(batch, seq, D)


# speedup vs baseline: 1.0000x; 1.0000x over previous
import functools
import math

import jax
import jax.numpy as jnp
import numpy as np
from jax import lax
from jax.experimental import pallas as pl
from jax.experimental.pallas import tpu as pltpu
from jax.experimental.pallas import tpu_sc as plsc

D = 64
SEQ = 50
L = 16
V = 1000000
VB = 999936
WV = 512
NWIN = 1954
NBP = 1968
FLAT = 204800
KPT = FLAT // 16
BKN = 221184
TRASH_B = 220000
OUT_ROWS = FLAT + 8
CHUNK = 128


def _pos_encoding(length: int, depth: int) -> np.ndarray:
    half = depth / 2
    positions = np.arange(length)[:, np.newaxis]
    depths = np.arange(half)[np.newaxis, :] / half
    angle_rates = 1 / 10000**depths
    angle_rads = positions * angle_rates
    return np.concatenate(
        [np.sin(angle_rads), np.cos(angle_rads)], axis=-1
    ).astype(np.float32)


_POS = _pos_encoding(SEQ, D)


@jax.jit
def _embed_sc(table_t, idx, pos1d, tail1d):
    mesh = plsc.VectorSubcoreMesh(core_axis_name="c", subcore_axis_name="s")

    @functools.partial(
        pl.kernel,
        out_type=jax.ShapeDtypeStruct((OUT_ROWS, 128), jnp.float32),
        mesh=mesh,
        scratch_types=[
            pltpu.VMEM((8, 4, 8, 128), jnp.float32),
            pltpu.VMEM((CHUNK, 128), jnp.float32),
            pltpu.VMEM((KPT,), jnp.int32),
            pltpu.VMEM((1600,), jnp.int32),
            pltpu.VMEM((1600,), jnp.int32),
            pltpu.VMEM((1600,), jnp.int32),
            pltpu.VMEM((NBP,), jnp.int32),
            pltpu.VMEM((NBP,), jnp.int32),
            pltpu.VMEM((NBP,), jnp.int32),
            pltpu.VMEM((NBP,), jnp.int32),
            pltpu.VMEM((NBP,), jnp.int32),
            pltpu.VMEM((NBP,), jnp.int32),
            pltpu.VMEM((SEQ * D,), jnp.float32),
            pltpu.VMEM((CHUNK,), jnp.int32),
            pltpu.VMEM((CHUNK,), jnp.int32),
            pltpu.VMEM_SHARED((16, NBP), jnp.int32),
            pltpu.VMEM_SHARED((BKN,), jnp.int32),
            pltpu.VMEM_SHARED((BKN,), jnp.int32),
            pltpu.SemaphoreType.DMA,
        ],
        compiler_params=pltpu.CompilerParams(needs_layout_passes=False),
    )
    def body(tt_h, idx_h, pos_h, tail_h, out_h,
             slab, ostage, midx, sc_d, sc_i, sc_k,
             cnt, tot, ownp, starts, woff, rowb,
             posv, ci, ck, cnt_sh, bk_i, bk_k, sem):
        t = lax.axis_index("s")
        c = lax.axis_index("c")
        wid = t * 2 + c
        lane = lax.iota(jnp.int32, 16)

        pltpu.sync_copy(pos_h, posv)
        pltpu.sync_copy(idx_h.at[pl.ds(t * KPT, KPT)], midx)

        @pl.loop(0, NBP // L)
        def _(cb):
            z = jnp.zeros((L,), jnp.int32)
            cnt[pl.ds(cb * L, L)] = z
            tot[pl.ds(cb * L, L)] = z
            ownp[pl.ds(cb * L, L)] = z

        @pl.loop(0, KPT // L)
        def _(g):
            iv = midx[pl.ds(g * L, L)]
            wv = iv >> 9
            rank, last = plsc.scan_count(wv)
            plsc.addupdate_scatter(cnt, [wv], rank, mask=last)

        pltpu.sync_copy(cnt, cnt_sh.at[t])
        plsc.subcore_barrier()

        for tt in range(16):
            pltpu.sync_copy(cnt_sh.at[tt], rowb)

            @pl.loop(0, NBP // L)
            def _(cb):
                r = rowb[pl.ds(cb * L, L)]
                tot[pl.ds(cb * L, L)] += r
                ownp[pl.ds(cb * L, L)] += jnp.where(t > tt, r, 0)

        @pl.loop(0, NBP // L, init_carry=jnp.int32(0))
        def _(cb, carry):
            t8 = (tot[pl.ds(cb * L, L)] + 7) & ~7
            cum = plsc.cumsum(t8)
            starts[pl.ds(cb * L, L)] = cum - t8 + carry
            return carry + jnp.sum(t8)

        @pl.loop(0, NBP // L)
        def _(cb):
            woff[pl.ds(cb * L, L)] = starts[pl.ds(cb * L, L)] + ownp[pl.ds(cb * L, L)]

        for blk in range(8):
            @pl.loop(0, 100)
            def _(gg):
                g = blk * 100 + gg
                iv = midx[pl.ds(g * L, L)]
                wv = iv >> 9
                rank, last = plsc.scan_count(wv)
                dest = plsc.load_gather(woff, [wv]) + rank - 1
                y = g * L + lane
                s = y - 50 * ((y * 5243) >> 18)
                sc_d[pl.ds(gg * L, L)] = dest
                sc_i[pl.ds(gg * L, L)] = iv
                sc_k[pl.ds(gg * L, L)] = (t * KPT + y) | (s << 18)
                plsc.addupdate_scatter(woff, [wv], rank, mask=last)

            pltpu.sync_copy(sc_i, bk_i.at[sc_d])
            pltpu.sync_copy(sc_k, bk_k.at[sc_d])

        plsc.subcore_barrier()

        @pl.loop(0, 62)
        def _(jj):
            w = wid + 32 * jj

            @pl.when(w < NWIN)
            def _():
                @pl.when(w < 1953)
                def _():
                    copies = []
                    for o in range(8):
                        for j in range(4):
                            copies.append(pltpu.async_copy(
                                tt_h.at[pl.ds(o * 8, 8),
                                        pl.ds((w * 4 + j) * 128, 128)],
                                slab.at[o, j], sem))
                    for cp in copies:
                        cp.wait()

                @pl.when(w == 1953)
                def _():
                    pltpu.sync_copy(tail_h, slab)

                wb = pl.multiple_of((w >> 4) << 4, L)
                sts = starts[pl.ds(wb, L)]
                tts = tot[pl.ds(wb, L)]
                sel = lane == (w - wb)
                st = pl.multiple_of(jnp.sum(jnp.where(sel, sts, 0)), 8)
                n = jnp.sum(jnp.where(sel, tts, 0))

                @pl.loop(0, (n + CHUNK - 1) >> 7)
                def _(ch):
                    pltpu.sync_copy(bk_i.at[pl.ds(st + ch * CHUNK, CHUNK)], ci)
                    pltpu.sync_copy(bk_k.at[pl.ds(st + ch * CHUNK, CHUNK)], ck)

                    @pl.loop(0, CHUNK // L)
                    def _(m):
                        iv = ci[pl.ds(m * L, L)]
                        kp = ck[pl.ds(m * L, L)]
                        kv = kp & 0x3FFFF
                        sv = jnp.minimum((kp >> 18) & 63, SEQ - 1)
                        valid = (m * L + lane) < (n - ch * CHUNK)
                        ivc = jnp.clip(iv, w * WV, w * WV + WV - 1)
                        ck[pl.ds(m * L, L)] = jnp.where(valid, kv, FLAT)
                        jv = (ivc - w * WV) >> 7
                        cv = ivc & 127
                        rowv = m * L + lane
                        for o in range(8):
                            ov = jnp.full((L,), o, jnp.int32)
                            for r in range(8):
                                dd = o * 8 + r
                                rv = jnp.full((L,), r, jnp.int32)
                                pv = plsc.load_gather(posv, [sv * D + dd])
                                tv = plsc.load_gather(slab, [ov, jv, rv, cv])
                                plsc.store_scatter(
                                    ostage, [rowv, jnp.full((L,), dd, jnp.int32)],
                                    tv * 8.0 + pv)

                    pltpu.sync_copy(ostage, out_h.at[ck])

    return body(table_t, idx, pos1d, tail1d)


def kernel(x, table):
    batch, seq = x.shape
    assert seq == SEQ and table.shape == (V, D)
    idx = x.reshape(-1).astype(jnp.int32)
    table_t = table.T
    tail4 = jnp.zeros((8, 4, 8, 128), jnp.float32).at[:, 0, :, :64].set(
        table[VB:, :].T.reshape(8, 8, 64))
    pos1d = jnp.asarray(_POS).reshape(-1)
    out128 = _embed_sc(table_t, idx, pos1d, tail4)
    return out128[:FLAT, :D].reshape(batch, seq, D)


# speedup vs baseline: 3.1571x; 3.1571x over previous
import functools
import math

import jax
import jax.numpy as jnp
import numpy as np
from jax import lax
from jax.experimental import pallas as pl
from jax.experimental.pallas import tpu as pltpu
from jax.experimental.pallas import tpu_sc as plsc

D_MODEL = 64
SEQ = 50
LANES = 16
NUM_WORKERS = 32


def _pos_encoding(length: int, depth: int) -> np.ndarray:
    half = depth / 2
    positions = np.arange(length)[:, np.newaxis]
    depths = np.arange(half)[np.newaxis, :] / half
    angle_rates = 1 / 10000**depths
    angle_rads = positions * angle_rates
    return np.concatenate(
        [np.sin(angle_rads), np.cos(angle_rads)], axis=-1
    ).astype(np.float32)


@functools.partial(jax.jit, static_argnames=("flat", "chunk"))
def _embed_sc(table, idx, pos, *, flat: int, chunk: int):
    rows_per_worker = flat // NUM_WORKERS
    n_chunks = rows_per_worker // chunk
    d_blocks = D_MODEL // LANES
    groups = chunk // SEQ

    mesh = plsc.VectorSubcoreMesh(core_axis_name="c", subcore_axis_name="s")

    @functools.partial(
        pl.kernel,
        out_type=jax.ShapeDtypeStruct((flat, D_MODEL), jnp.float32),
        mesh=mesh,
        scratch_types=[
            pltpu.VMEM((chunk,), jnp.int32),
            pltpu.VMEM((chunk, D_MODEL), jnp.float32),
            pltpu.VMEM((SEQ, D_MODEL), jnp.float32),
            pltpu.SemaphoreType.DMA,
        ],
        compiler_params=pltpu.CompilerParams(use_tc_tiling_on_sc=False),
    )
    def body(table_hbm, idx_hbm, pos_hbm, out_hbm, idx_v, rows_v, pos_v, sem):
        wid = lax.axis_index("s") * 2 + lax.axis_index("c")
        base = wid * rows_per_worker
        pltpu.sync_copy(pos_hbm, pos_v)

        for c in range(n_chunks):
            start = base + c * chunk
            pltpu.sync_copy(idx_hbm.at[pl.ds(start, chunk)], idx_v)
            pltpu.async_copy(table_hbm.at[idx_v], rows_v, sem).wait()

            @pl.loop(0, SEQ)
            def _(p):
                for d in range(d_blocks):
                    pe = pos_v[p, pl.ds(d * LANES, LANES)]
                    for g in range(groups):
                        j = p + g * SEQ
                        v = rows_v[j, pl.ds(d * LANES, LANES)]
                        rows_v[j, pl.ds(d * LANES, LANES)] = v * 8.0 + pe

            pltpu.sync_copy(rows_v, out_hbm.at[pl.ds(start, chunk)])

    return body(table, idx, pos)


_POS = _pos_encoding(SEQ, D_MODEL)


def kernel(x, table):
    batch, seq = x.shape
    assert seq == SEQ and table.shape[1] == D_MODEL
    flat = batch * seq
    idx = x.reshape(flat).astype(jnp.int32)
    pos = jnp.asarray(_POS)
    out = _embed_sc(table, idx, pos, flat=flat, chunk=800)
    return out.reshape(batch, seq, D_MODEL)


# speedup vs baseline: 3.2128x; 1.0176x over previous
import functools

import jax
import jax.numpy as jnp
import numpy as np
from jax import lax
from jax.experimental import pallas as pl
from jax.experimental.pallas import tpu as pltpu
from jax.experimental.pallas import tpu_sc as plsc

D_MODEL = 64
SEQ = 50
LANES = 16
NUM_WORKERS = 32


def _pos_encoding(length: int, depth: int) -> np.ndarray:
    half = depth / 2
    positions = np.arange(length)[:, np.newaxis]
    depths = np.arange(half)[np.newaxis, :] / half
    angle_rates = 1 / 10000**depths
    angle_rads = positions * angle_rates
    return np.concatenate(
        [np.sin(angle_rads), np.cos(angle_rads)], axis=-1
    ).astype(np.float32)


@functools.partial(jax.jit, static_argnames=("flat", "chunk"))
def _embed_sc(table, idx, pos, *, flat: int, chunk: int):
    rows_per_worker = flat // NUM_WORKERS
    n_chunks = rows_per_worker // chunk
    d_blocks = D_MODEL // LANES
    groups = chunk // SEQ

    mesh = plsc.VectorSubcoreMesh(core_axis_name="c", subcore_axis_name="s")

    @functools.partial(
        pl.kernel,
        out_type=jax.ShapeDtypeStruct((flat, D_MODEL), jnp.float32),
        mesh=mesh,
        scratch_types=[
            pltpu.VMEM((chunk,), jnp.int32),
            pltpu.VMEM((chunk,), jnp.int32),
            pltpu.VMEM((chunk, D_MODEL), jnp.float32),
            pltpu.VMEM((chunk, D_MODEL), jnp.float32),
            pltpu.VMEM((SEQ, D_MODEL), jnp.float32),
            pltpu.SemaphoreType.DMA,
            pltpu.SemaphoreType.DMA,
            pltpu.SemaphoreType.DMA,
            pltpu.SemaphoreType.DMA,
        ],
        compiler_params=pltpu.CompilerParams(use_tc_tiling_on_sc=False),
    )
    def body(table_hbm, idx_hbm, pos_hbm, out_hbm,
             idx0, idx1, rows0, rows1, pos_v, g0, g1, o0, o1):
        wid = lax.axis_index("s") * 2 + lax.axis_index("c")
        base = wid * rows_per_worker
        idx_v = (idx0, idx1)
        rows_v = (rows0, rows1)
        gsem = (g0, g1)
        osem = (o0, o1)
        pltpu.sync_copy(pos_hbm, pos_v)

        gathers = [None, None]
        outs = [None, None]

        def start_gather(c):
            b = c % 2
            start = base + c * chunk
            pltpu.sync_copy(idx_hbm.at[pl.ds(start, chunk)], idx_v[b])
            gathers[b] = pltpu.async_copy(
                table_hbm.at[idx_v[b]], rows_v[b], gsem[b])

        start_gather(0)
        for c in range(n_chunks):
            b = c % 2
            nb = (c + 1) % 2
            if c + 1 < n_chunks:
                if outs[nb] is not None:
                    outs[nb].wait()
                    outs[nb] = None
                start_gather(c + 1)
            gathers[b].wait()

            rv = rows_v[b]

            @pl.loop(0, SEQ)
            def _(p):
                for d in range(d_blocks):
                    pe = pos_v[p, pl.ds(d * LANES, LANES)]
                    for g in range(groups):
                        j = p + g * SEQ
                        v = rv[j, pl.ds(d * LANES, LANES)]
                        rv[j, pl.ds(d * LANES, LANES)] = v * 8.0 + pe

            start = base + c * chunk
            outs[b] = pltpu.async_copy(
                rows_v[b], out_hbm.at[pl.ds(start, chunk)], osem[b])

        for b in range(2):
            if outs[b] is not None:
                outs[b].wait()

    return body(table, idx, pos)


_POS = _pos_encoding(SEQ, D_MODEL)


def kernel(x, table):
    batch, seq = x.shape
    assert seq == SEQ and table.shape[1] == D_MODEL
    flat = batch * seq
    idx = x.reshape(flat).astype(jnp.int32)
    pos = jnp.asarray(_POS)
    out = _embed_sc(table, idx, pos, flat=flat, chunk=800)
    return out.reshape(batch, seq, D_MODEL)
